# baseline (speedup 1.0000x reference)
import functools

import jax
import jax.numpy as jnp
from jax import lax
from jax.experimental import pallas as pl
from jax.experimental.pallas import tpu as pltpu
from jax.experimental.pallas import tpu_sc as plsc

D = 128
DI = 64
L = 16
NC, NS = 2, 16
NW = NC * NS
E = 320000
EPW = E // NW
C = 80
NCHUNK = EPW // C


def _decoder_body(z_hbm, src_hbm, dst_hbm, out_hbm,
                  sidx_all, didx_all,
                  srow0, drow0, srow1, drow1,
                  out_v, sem_s0, sem_d0, sem_s1, sem_d1):
    wid = lax.axis_index("s") * NC + lax.axis_index("c")
    base = wid * EPW
    lane = lax.iota(jnp.int32, L)

    srow = (srow0, srow1)
    drow = (drow0, drow1)
    sem_s = (sem_s0, sem_s1)
    sem_d = (sem_d0, sem_d1)

    pltpu.sync_copy(src_hbm.at[pl.ds(base, EPW)], sidx_all)
    pltpu.sync_copy(dst_hbm.at[pl.ds(base, EPW)], didx_all)

    def issue(g, b):
        pltpu.async_copy(z_hbm.at[sidx_all.at[pl.ds(g * C, C)]], srow[b], sem_s[b])
        pltpu.async_copy(z_hbm.at[didx_all.at[pl.ds(g * C, C)]], drow[b], sem_d[b])

    def wait(g, b):
        pltpu.make_async_copy(
            z_hbm.at[sidx_all.at[pl.ds(g * C, C)]], srow[b], sem_s[b]).wait()
        pltpu.make_async_copy(
            z_hbm.at[didx_all.at[pl.ds(g * C, C)]], drow[b], sem_d[b]).wait()

    def compute(g, b):
        sr, dr = srow[b], drow[b]

        def group_body(t, carry2):
            rows = t * L + lane
            accs = [jnp.zeros((L,), jnp.float32) for _ in range(4)]
            for j in range(DI):
                cols = jnp.full((L,), j, jnp.int32)
                a = plsc.bitcast(plsc.load_gather(sr, [rows, cols]), jnp.bfloat16)
                b_ = plsc.bitcast(plsc.load_gather(dr, [rows, cols]), jnp.bfloat16)
                p0, p1 = plsc.unpack(a * b_, format=plsc.PackFormat.INTERLEAVED)
                accs[2 * (j % 2)] += p0
                accs[2 * (j % 2) + 1] += p1
            vec = (accs[0] + accs[1]) + (accs[2] + accs[3])
            out_v[pl.ds(g * C + t * L, L)] = 1.0 / (1.0 + jnp.exp(-vec))
            return carry2

        lax.fori_loop(0, C // L, group_body, 0)

    issue(0, 0)

    def pair_body(i, carry):
        for b in range(2):
            g = 2 * i + b
            wait(g, b)
            issue(g + 1, 1 - b)
            compute(g, b)
        return carry

    lax.fori_loop(0, (NCHUNK - 1) // 2, pair_body, 0)
    wait(NCHUNK - 1, 0)
    compute(NCHUNK - 1, 0)

    pltpu.sync_copy(out_v, out_hbm.at[pl.ds(base, EPW)])


_decoder = functools.partial(
    pl.kernel,
    out_type=jax.ShapeDtypeStruct((E,), jnp.float32),
    mesh=plsc.VectorSubcoreMesh(core_axis_name="c", subcore_axis_name="s"),
    compiler_params=pltpu.CompilerParams(
        needs_layout_passes=False, use_tc_tiling_on_sc=False),
    scratch_types=[
        pltpu.VMEM((EPW,), jnp.int32),
        pltpu.VMEM((EPW,), jnp.int32),
        pltpu.VMEM((C, DI), jnp.int32),
        pltpu.VMEM((C, DI), jnp.int32),
        pltpu.VMEM((C, DI), jnp.int32),
        pltpu.VMEM((C, DI), jnp.int32),
        pltpu.VMEM((EPW,), jnp.float32),
        pltpu.SemaphoreType.DMA,
        pltpu.SemaphoreType.DMA,
        pltpu.SemaphoreType.DMA,
        pltpu.SemaphoreType.DMA,
    ],
)(_decoder_body)


def kernel(z, edge_index):
    ei = edge_index.astype(jnp.int32)
    zb = z.astype(jnp.bfloat16)
    zi = jax.lax.bitcast_convert_type(zb.reshape(z.shape[0], DI, 2), jnp.int32)
    return _decoder(zi, ei[0], ei[1])


# speedup vs baseline: 4.2938x; 4.2938x over previous
import functools

import jax
import jax.numpy as jnp
from jax import lax
from jax.experimental import pallas as pl
from jax.experimental.pallas import tpu as pltpu
from jax.experimental.pallas import tpu_sc as plsc

D = 128
DI = 64
L = 16
NC, NS = 2, 16
NW = NC * NS
E = 320000
EPW = E // NW
C = 80
NCHUNK = EPW // C


def _decoder_body(z_hbm, src_hbm, dst_hbm, out_hbm,
                  sidx_all, didx_all,
                  srow0, drow0, srow1, drow1,
                  out_v, sem_s0, sem_d0, sem_s1, sem_d1):
    wid = lax.axis_index("s") * NC + lax.axis_index("c")
    base = wid * EPW
    lane = lax.iota(jnp.int32, L)

    srow = (srow0, srow1)
    drow = (drow0, drow1)
    sem_s = (sem_s0, sem_s1)
    sem_d = (sem_d0, sem_d1)

    pltpu.sync_copy(src_hbm.at[pl.ds(base, EPW)], sidx_all)
    pltpu.sync_copy(dst_hbm.at[pl.ds(base, EPW)], didx_all)

    def issue(g, b):
        pltpu.async_copy(z_hbm.at[sidx_all.at[pl.ds(g * C, C)]], srow[b], sem_s[b])
        pltpu.async_copy(z_hbm.at[didx_all.at[pl.ds(g * C, C)]], drow[b], sem_d[b])

    def wait(g, b):
        pltpu.make_async_copy(
            z_hbm.at[sidx_all.at[pl.ds(g * C, C)]], srow[b], sem_s[b]).wait()
        pltpu.make_async_copy(
            z_hbm.at[didx_all.at[pl.ds(g * C, C)]], drow[b], sem_d[b]).wait()

    def compute(g, b):
        sr, dr = srow[b], drow[b]

        def group_body(t, carry2):
            vec = jnp.zeros((L,), jnp.float32)
            for k in range(L):
                e = t * L + k
                acc0 = jnp.zeros((L,), jnp.float32)
                acc1 = jnp.zeros((L,), jnp.float32)
                for j in range(DI // L):
                    a = plsc.bitcast(sr[e, pl.ds(j * L, L)], jnp.bfloat16)
                    b_ = plsc.bitcast(dr[e, pl.ds(j * L, L)], jnp.bfloat16)
                    p0, p1 = plsc.unpack(a * b_, format=plsc.PackFormat.INTERLEAVED)
                    acc0 += p0
                    acc1 += p1
                vec = jnp.where(lane == k, jnp.sum(acc0 + acc1), vec)
            out_v[pl.ds(g * C + t * L, L)] = 1.0 / (1.0 + jnp.exp(-vec))
            return carry2

        lax.fori_loop(0, C // L, group_body, 0)

    issue(0, 0)

    def pair_body(i, carry):
        for b in range(2):
            g = 2 * i + b
            wait(g, b)
            issue(g + 1, 1 - b)
            compute(g, b)
        return carry

    lax.fori_loop(0, (NCHUNK - 1) // 2, pair_body, 0)
    wait(NCHUNK - 1, 0)
    compute(NCHUNK - 1, 0)

    pltpu.sync_copy(out_v, out_hbm.at[pl.ds(base, EPW)])


_decoder = functools.partial(
    pl.kernel,
    out_type=jax.ShapeDtypeStruct((E,), jnp.float32),
    mesh=plsc.VectorSubcoreMesh(core_axis_name="c", subcore_axis_name="s"),
    compiler_params=pltpu.CompilerParams(
        needs_layout_passes=False, use_tc_tiling_on_sc=False),
    scratch_types=[
        pltpu.VMEM((EPW,), jnp.int32),
        pltpu.VMEM((EPW,), jnp.int32),
        pltpu.VMEM((C, DI), jnp.int32),
        pltpu.VMEM((C, DI), jnp.int32),
        pltpu.VMEM((C, DI), jnp.int32),
        pltpu.VMEM((C, DI), jnp.int32),
        pltpu.VMEM((EPW,), jnp.float32),
        pltpu.SemaphoreType.DMA,
        pltpu.SemaphoreType.DMA,
        pltpu.SemaphoreType.DMA,
        pltpu.SemaphoreType.DMA,
    ],
)(_decoder_body)


def kernel(z, edge_index):
    ei = edge_index.astype(jnp.int32)
    zb = z.astype(jnp.bfloat16)
    zi = jax.lax.bitcast_convert_type(zb.reshape(z.shape[0], DI, 2), jnp.int32)
    return _decoder(zi, ei[0], ei[1])
